# baseline (speedup 1.0000x reference)
import jax
import jax.numpy as jnp
from jax import lax
from jax.experimental import pallas as pl
from jax.experimental.pallas import tpu as pltpu
from jax.experimental.pallas import tpu_sc as plsc

S = 8
R = 128
IN_DIM = 32
HID = 32
N = 262144
D2 = 2 * IN_DIM

NC = 2
NS = 16
NW = NC * NS
PPW = N // NW
CH = 64
NCHUNK = PPW // CH


def _prep_body(b_ref, px_ref, py_ref, pz_ref, iw_ref):
    px = px_ref[...]
    py = py_ref[...]
    pz = pz_ref[...]
    shp = px.shape
    pre = jnp.zeros(shp, jnp.bool_)
    lox = jnp.zeros(shp, jnp.float32)
    loy = jnp.zeros(shp, jnp.float32)
    loz = jnp.zeros(shp, jnp.float32)
    hix = jnp.ones(shp, jnp.float32)
    hiy = jnp.ones(shp, jnp.float32)
    hiz = jnp.ones(shp, jnp.float32)
    sidx = jnp.zeros(shp, jnp.int32)
    for s in range(S):
        l0 = b_ref[s, 0, 0]
        l1 = b_ref[s, 0, 1]
        l2 = b_ref[s, 0, 2]
        h0 = b_ref[s, 1, 0]
        h1 = b_ref[s, 1, 1]
        h2 = b_ref[s, 1, 2]
        m = ((px > l0) & (px < h0) & (py > l1) & (py < h1)
             & (pz > l2) & (pz < h2) & (~pre))
        pre = pre | m
        lox = jnp.where(m, l0, lox)
        loy = jnp.where(m, l1, loy)
        loz = jnp.where(m, l2, loz)
        hix = jnp.where(m, h0, hix)
        hiy = jnp.where(m, h1, hiy)
        hiz = jnp.where(m, h2, hiz)
        sidx = jnp.where(m, s, sidx)
    routed = pre.astype(jnp.float32)
    dx = jnp.where(pre, hix - lox, 1.0)
    dy = jnp.where(pre, hiy - loy, 1.0)
    dz = jnp.where(pre, hiz - loz, 1.0)
    un = (px - lox) / dx
    vn = (py - loy) / dy
    tn = (pz - loz) / dz
    sbase = sidx * (R * R)
    for o, (ca, cb) in enumerate(((un, vn), (un, tn), (vn, tn))):
        xx = jnp.clip(ca, 0.0, 1.0) * (R - 1)
        yy = jnp.clip(cb, 0.0, 1.0) * (R - 1)
        x0 = jnp.clip(jnp.floor(xx), 0, R - 2).astype(jnp.int32)
        y0 = jnp.clip(jnp.floor(yy), 0, R - 2).astype(jnp.int32)
        wx = xx - x0.astype(jnp.float32)
        wy = yy - y0.astype(jnp.float32)
        base = sbase + x0 * R + y0
        iw_ref[4 * o + 0] = base
        iw_ref[4 * o + 1] = base + 1
        iw_ref[4 * o + 2] = base + R
        iw_ref[4 * o + 3] = base + R + 1
        wq = ((1 - wx) * (1 - wy), (1 - wx) * wy,
              wx * (1 - wy), wx * wy)
        for q in range(4):
            iw_ref[12 + 4 * o + q] = lax.bitcast_convert_type(
                wq[q] * routed, jnp.int32)


def _prep(px, py, pz, boundaries):
    nb = px.shape[0]
    blk = 256
    grid = nb // blk
    return pl.pallas_call(
        _prep_body,
        grid=(grid,),
        in_specs=[
            pl.BlockSpec(memory_space=pltpu.SMEM),
            pl.BlockSpec((blk, 128), lambda i: (i, 0)),
            pl.BlockSpec((blk, 128), lambda i: (i, 0)),
            pl.BlockSpec((blk, 128), lambda i: (i, 0)),
        ],
        out_specs=pl.BlockSpec((24, blk, 128), lambda i: (0, i, 0)),
        out_shape=jax.ShapeDtypeStruct((24, nb, 128), jnp.int32),
        compiler_params=pltpu.CompilerParams(
            allow_input_fusion=[False, True, True, True]),
    )(boundaries, px, py, pz)


def _sc_body(iw_hbm, t0, t1, t2, t3, t4, t5, feat_hbm, *scr):
    iw_v = scr[0:4]
    rows = (scr[4:28], scr[28:52])
    outb = scr[52:54]
    gsem = scr[54:56]
    osem = scr[56:58]
    iwsem = scr[58:62]
    tabs = (t0, t1, t2, t3, t4, t5)

    wid = lax.axis_index("s") * NC + lax.axis_index("c")
    base0 = wid * PPW
    iota16 = lax.iota(jnp.int32, 16)

    def iw_load(c, ib):
        base = pl.multiple_of(base0 + c * CH, CH)
        pltpu.sync_copy(iw_hbm.at[:, pl.ds(base, CH)], iw_v[ib])

    def iw_fire(c, ib):
        base = pl.multiple_of(base0 + c * CH, CH)
        pltpu.async_copy(iw_hbm.at[:, pl.ds(base, CH)], iw_v[ib],
                         iwsem[ib])

    def iw_drain(ib):
        pltpu.make_async_copy(iw_hbm.at[:, pl.ds(0, CH)], iw_v[ib],
                              iwsem[ib]).wait()

    def fire(b, ib):
        for j in range(24):
            pltpu.async_copy(tabs[3 * (j // 12) + (j % 12) // 4]
                             .at[iw_v[ib].at[j % 12]],
                             rows[b][j], gsem[b])

    def drain_gathers(b, ib):
        for j in range(24):
            pltpu.make_async_copy(tabs[3 * (j // 12) + (j % 12) // 4]
                                  .at[iw_v[ib].at[j % 12]],
                                  rows[b][j], gsem[b]).wait()

    kvecs = [iota16 + 16 * k for k in range(D2 // 16)]

    def compute(b, ib):
        def p_body(pi, carry):
            for u in range(2):
                p = 2 * pi + u
                pv = jnp.full((16,), 0, jnp.int32) + p
                wq = [plsc.bitcast(
                    plsc.load_gather(iw_v[ib],
                                     [jnp.full((16,), 12 + q, jnp.int32),
                                      pv]),
                    jnp.float32) for q in range(12)]
                accs = []
                for k in range(4):
                    half = 12 * (k // 2)
                    kv = kvecs[k % 2]
                    acc = None
                    for j in range(12):
                        t = wq[j] * plsc.load_gather(rows[b][half + j],
                                                    [pv, kv])
                        acc = t if acc is None else acc + t
                    accs.append(acc)
                for h in range(2):
                    pk = plsc.pack(accs[2 * h], accs[2 * h + 1],
                                   format=plsc.PackFormat.INTERLEAVED)
                    plsc.store_scatter(outb[b], [pv, kvecs[h]],
                                       plsc.bitcast(pk, jnp.int32))
            return carry

        lax.fori_loop(0, CH // 2, p_body, 0)

    def out_fire(c, b):
        base = pl.multiple_of(base0 + c * CH, CH)
        pltpu.async_copy(outb[b], feat_hbm.at[pl.ds(base, CH)], osem[b])

    def out_drain(b):
        pltpu.make_async_copy(outb[b], feat_hbm.at[pl.ds(0, CH)],
                              osem[b]).wait()

    iw_load(0, 0)
    iw_load(1, 1)
    iw_load(2, 2)
    fire(0, 0)

    def quad_body(i, carry):
        for u in range(4):
            c = 4 * i + u
            b = u % 2
            ib = u
            nc = c + 1

            @pl.when(nc < NCHUNK)
            def _():
                @pl.when(nc >= 3)
                def _():
                    iw_drain((u + 1) % 4)
                fire(1 - b, (u + 1) % 4)

            @pl.when(c + 3 < NCHUNK)
            def _():
                iw_fire(c + 3, (u + 3) % 4)

            drain_gathers(b, ib)

            @pl.when(c >= 2)
            def _():
                out_drain(b)

            compute(b, ib)
            out_fire(c, b)
        return carry

    lax.fori_loop(0, NCHUNK // 4, quad_body, 0)
    out_drain(0)
    out_drain(1)


def _gather_sc(iw, *tabs):
    mesh = plsc.VectorSubcoreMesh(
        core_axis_name="c", subcore_axis_name="s",
        num_cores=NC, num_subcores=NS)
    scratch = (
        [pltpu.VMEM((24, CH), jnp.int32) for _ in range(4)]
        + [pltpu.VMEM((CH, IN_DIM), jnp.float32) for _ in range(48)]
        + [pltpu.VMEM((CH, IN_DIM), jnp.int32) for _ in range(2)]
        + [pltpu.SemaphoreType.DMA for _ in range(8)]
    )
    fn = pl.kernel(
        _sc_body,
        out_type=jax.ShapeDtypeStruct((N, IN_DIM), jnp.int32),
        mesh=mesh,
        scratch_types=scratch,
        compiler_params=pltpu.CompilerParams(use_tc_tiling_on_sc=False,
                                             needs_layout_passes=False),
    )
    return fn(iw, *tabs)


def _mlp_body(f_ref, W0r, b0r, W1r, b1r, Wfr, bfr, out_ref):
    f = f_ref[...].astype(jnp.float32)
    h = jnp.maximum(jnp.dot(f, W0r[...], preferred_element_type=jnp.float32)
                    + b0r[...], 0.0)
    h = jnp.maximum(jnp.dot(h, W1r[...], preferred_element_type=jnp.float32)
                    + b1r[...], 0.0)
    z = jnp.dot(h, Wfr[...], preferred_element_type=jnp.float32) + bfr[...]
    col = lax.broadcasted_iota(jnp.int32, z.shape, 1)
    out_ref[...] = jnp.where(col < 3, jax.nn.sigmoid(z), jnp.tanh(z))


def _mlp(feat, W0c, b0c, W1c, b1c, Wf, bf):
    blk = 4096
    grid = N // blk

    def fullspec(a):
        return pl.BlockSpec(a.shape, lambda i: (0,) * a.ndim)

    ws = [W0c, b0c, W1c, b1c, Wf, bf]
    return pl.pallas_call(
        _mlp_body,
        grid=(grid,),
        in_specs=([pl.BlockSpec((blk, D2), lambda i: (i, 0))]
                  + [fullspec(a) for a in ws]),
        out_specs=pl.BlockSpec((blk, 4), lambda i: (i, 0)),
        out_shape=jax.ShapeDtypeStruct((N, 4), jnp.float32),
    )(feat, *ws)


def kernel(p, boundaries, planes_xy, planes_xz, planes_yz,
           c_planes_xy, c_planes_xz, c_planes_yz,
           W0, b0, W1, b1, Wout, bout, cW0, cb0, cW1, cb1, cWout, cbout):
    nb = N // 128
    px = p[:, 0].reshape(nb, 128)
    py = p[:, 1].reshape(nb, 128)
    pz = p[:, 2].reshape(nb, 128)
    iw3 = _prep(px, py, pz, boundaries)
    iw = iw3.reshape(24, N)

    tabs = [a.reshape(S * R * R, IN_DIM)
            for a in (planes_xy, planes_xz, planes_yz,
                      c_planes_xy, c_planes_xz, c_planes_yz)]
    feat_i = _gather_sc(iw, *tabs)
    feat = lax.bitcast_convert_type(feat_i, jnp.bfloat16).reshape(N, D2)

    perm = []
    for h in range(2):
        for i in range(HID // 2):
            perm += [HID * h + i, HID * h + HID // 2 + i]
    zz = jnp.zeros((HID, HID), jnp.float32)
    W0c = jnp.block([[W0, zz], [zz, cW0]])
    W0c = W0c[jnp.array(perm), :]
    b0c = jnp.concatenate([b0, cb0]).reshape(1, D2)
    W1c = jnp.block([[W1, zz], [zz, cW1]])
    b1c = jnp.concatenate([b1, cb1]).reshape(1, D2)
    Wf = jnp.block([[jnp.zeros((HID, 3), jnp.float32), Wout],
                    [cWout, jnp.zeros((HID, 1), jnp.float32)]])
    bf = jnp.concatenate([cbout, bout]).reshape(1, 4)
    return _mlp(feat, W0c, b0c, W1c, b1c, Wf, bf)


# speedup vs baseline: 1.0500x; 1.0500x over previous
import jax
import jax.numpy as jnp
from jax import lax
from jax.experimental import pallas as pl
from jax.experimental.pallas import tpu as pltpu
from jax.experimental.pallas import tpu_sc as plsc

S = 8
R = 128
IN_DIM = 32
HID = 32
N = 262144
D2 = 2 * IN_DIM

NC = 2
NS = 16
NW = NC * NS
PPW = N // NW
CH = 64
NCHUNK = PPW // CH


def _prep_body(b_ref, px_ref, py_ref, pz_ref, iw_ref):
    px = px_ref[...]
    py = py_ref[...]
    pz = pz_ref[...]
    shp = px.shape
    pre = jnp.zeros(shp, jnp.bool_)
    lox = jnp.zeros(shp, jnp.float32)
    loy = jnp.zeros(shp, jnp.float32)
    loz = jnp.zeros(shp, jnp.float32)
    hix = jnp.ones(shp, jnp.float32)
    hiy = jnp.ones(shp, jnp.float32)
    hiz = jnp.ones(shp, jnp.float32)
    sidx = jnp.zeros(shp, jnp.int32)
    for s in range(S):
        l0 = b_ref[s, 0, 0]
        l1 = b_ref[s, 0, 1]
        l2 = b_ref[s, 0, 2]
        h0 = b_ref[s, 1, 0]
        h1 = b_ref[s, 1, 1]
        h2 = b_ref[s, 1, 2]
        m = ((px > l0) & (px < h0) & (py > l1) & (py < h1)
             & (pz > l2) & (pz < h2) & (~pre))
        pre = pre | m
        lox = jnp.where(m, l0, lox)
        loy = jnp.where(m, l1, loy)
        loz = jnp.where(m, l2, loz)
        hix = jnp.where(m, h0, hix)
        hiy = jnp.where(m, h1, hiy)
        hiz = jnp.where(m, h2, hiz)
        sidx = jnp.where(m, s, sidx)
    routed = pre.astype(jnp.float32)
    dx = jnp.where(pre, hix - lox, 1.0)
    dy = jnp.where(pre, hiy - loy, 1.0)
    dz = jnp.where(pre, hiz - loz, 1.0)
    un = (px - lox) / dx
    vn = (py - loy) / dy
    tn = (pz - loz) / dz
    sbase = sidx * (R * R)
    for o, (ca, cb) in enumerate(((un, vn), (un, tn), (vn, tn))):
        xx = jnp.clip(ca, 0.0, 1.0) * (R - 1)
        yy = jnp.clip(cb, 0.0, 1.0) * (R - 1)
        x0 = jnp.clip(jnp.floor(xx), 0, R - 2).astype(jnp.int32)
        y0 = jnp.clip(jnp.floor(yy), 0, R - 2).astype(jnp.int32)
        wx = xx - x0.astype(jnp.float32)
        wy = yy - y0.astype(jnp.float32)
        base = sbase + x0 * R + y0
        iw_ref[4 * o + 0] = base
        iw_ref[4 * o + 1] = base + 1
        iw_ref[4 * o + 2] = base + R
        iw_ref[4 * o + 3] = base + R + 1
        wq = ((1 - wx) * (1 - wy), (1 - wx) * wy,
              wx * (1 - wy), wx * wy)
        for q in range(4):
            iw_ref[12 + 4 * o + q] = lax.bitcast_convert_type(
                wq[q] * routed, jnp.int32)


def _prep(px, py, pz, boundaries):
    nb = px.shape[0]
    blk = 256
    grid = nb // blk
    return pl.pallas_call(
        _prep_body,
        grid=(grid,),
        in_specs=[
            pl.BlockSpec(memory_space=pltpu.SMEM),
            pl.BlockSpec((blk, 128), lambda i: (i, 0)),
            pl.BlockSpec((blk, 128), lambda i: (i, 0)),
            pl.BlockSpec((blk, 128), lambda i: (i, 0)),
        ],
        out_specs=pl.BlockSpec((24, blk, 128), lambda i: (0, i, 0)),
        out_shape=jax.ShapeDtypeStruct((24, nb, 128), jnp.int32),
        compiler_params=pltpu.CompilerParams(
            allow_input_fusion=[False, True, True, True]),
    )(boundaries, px, py, pz)


def _sc_body(iw_hbm, t0, t1, t2, t3, t4, t5, feat_hbm, *scr):
    iw_v = scr[0:4]
    rows = (scr[4:28], scr[28:52])
    outb = scr[52:54]
    gsem = scr[54:56]
    osem = scr[56:58]
    iwsem = scr[58:62]
    tabs = (t0, t1, t2, t3, t4, t5)

    wid = lax.axis_index("s") * NC + lax.axis_index("c")
    base0 = wid * PPW
    iota16 = lax.iota(jnp.int32, 16)

    def iw_load(c, ib):
        base = pl.multiple_of(base0 + c * CH, CH)
        pltpu.sync_copy(iw_hbm.at[:, pl.ds(base, CH)], iw_v[ib])

    def iw_fire(c, ib):
        base = pl.multiple_of(base0 + c * CH, CH)
        pltpu.async_copy(iw_hbm.at[:, pl.ds(base, CH)], iw_v[ib],
                         iwsem[ib])

    def iw_drain(ib):
        pltpu.make_async_copy(iw_hbm.at[:, pl.ds(0, CH)], iw_v[ib],
                              iwsem[ib]).wait()

    def fire(b, ib):
        for j in range(24):
            pltpu.async_copy(tabs[3 * (j // 12) + (j % 12) // 4]
                             .at[iw_v[ib].at[j % 12]],
                             rows[b][j], gsem[b])

    def drain_gathers(b, ib):
        for j in range(24):
            pltpu.make_async_copy(tabs[3 * (j // 12) + (j % 12) // 4]
                                  .at[iw_v[ib].at[j % 12]],
                                  rows[b][j], gsem[b]).wait()

    kvecs = [iota16 + 16 * k for k in range(D2 // 16)]

    def compute(b, ib):
        def p_body(pi, carry):
            for u in range(2):
                p = 2 * pi + u
                pv = jnp.full((16,), 0, jnp.int32) + p
                wq = [plsc.bitcast(
                    plsc.load_gather(iw_v[ib],
                                     [jnp.full((16,), 12 + q, jnp.int32),
                                      pv]),
                    jnp.float32) for q in range(12)]
                for k in range(4):
                    half = 12 * (k // 2)
                    kv = kvecs[k % 2]
                    acc = None
                    for j in range(12):
                        t = wq[j] * plsc.load_gather(rows[b][half + j],
                                                    [pv, kv])
                        acc = t if acc is None else acc + t
                    plsc.store_scatter(outb[b], [pv, kvecs[k]], acc)
            return carry

        lax.fori_loop(0, CH // 2, p_body, 0)

    def out_fire(c, b):
        base = pl.multiple_of(base0 + c * CH, CH)
        pltpu.async_copy(outb[b], feat_hbm.at[pl.ds(base, CH)], osem[b])

    def out_drain(b):
        pltpu.make_async_copy(outb[b], feat_hbm.at[pl.ds(0, CH)],
                              osem[b]).wait()

    iw_load(0, 0)
    iw_load(1, 1)
    iw_load(2, 2)
    fire(0, 0)

    def quad_body(i, carry):
        for u in range(4):
            c = 4 * i + u
            b = u % 2
            ib = u
            nc = c + 1

            @pl.when(nc < NCHUNK)
            def _():
                @pl.when(nc >= 3)
                def _():
                    iw_drain((u + 1) % 4)
                fire(1 - b, (u + 1) % 4)

            @pl.when(c + 3 < NCHUNK)
            def _():
                iw_fire(c + 3, (u + 3) % 4)

            drain_gathers(b, ib)

            @pl.when(c >= 2)
            def _():
                out_drain(b)

            compute(b, ib)
            out_fire(c, b)
        return carry

    lax.fori_loop(0, NCHUNK // 4, quad_body, 0)
    out_drain(0)
    out_drain(1)


def _gather_sc(iw, *tabs):
    mesh = plsc.VectorSubcoreMesh(
        core_axis_name="c", subcore_axis_name="s",
        num_cores=NC, num_subcores=NS)
    scratch = (
        [pltpu.VMEM((24, CH), jnp.int32) for _ in range(4)]
        + [pltpu.VMEM((CH, IN_DIM), jnp.float32) for _ in range(48)]
        + [pltpu.VMEM((CH, D2), jnp.float32) for _ in range(2)]
        + [pltpu.SemaphoreType.DMA for _ in range(8)]
    )
    fn = pl.kernel(
        _sc_body,
        out_type=jax.ShapeDtypeStruct((N, D2), jnp.float32),
        mesh=mesh,
        scratch_types=scratch,
        compiler_params=pltpu.CompilerParams(use_tc_tiling_on_sc=False,
                                             needs_layout_passes=False),
    )
    return fn(iw, *tabs)


def _mlp_body(f_ref, W0r, b0r, W1r, b1r, Wfr, bfr, out_ref):
    f = f_ref[...]
    h = jnp.maximum(jnp.dot(f, W0r[...], preferred_element_type=jnp.float32)
                    + b0r[...], 0.0)
    h = jnp.maximum(jnp.dot(h, W1r[...], preferred_element_type=jnp.float32)
                    + b1r[...], 0.0)
    z = jnp.dot(h, Wfr[...], preferred_element_type=jnp.float32) + bfr[...]
    col = lax.broadcasted_iota(jnp.int32, z.shape, 1)
    out_ref[...] = jnp.where(col < 3, jax.nn.sigmoid(z), jnp.tanh(z))


def _mlp(feat, W0c, b0c, W1c, b1c, Wf, bf):
    blk = 4096
    grid = N // blk

    def fullspec(a):
        return pl.BlockSpec(a.shape, lambda i: (0,) * a.ndim)

    ws = [W0c, b0c, W1c, b1c, Wf, bf]
    return pl.pallas_call(
        _mlp_body,
        grid=(grid,),
        in_specs=([pl.BlockSpec((blk, D2), lambda i: (i, 0))]
                  + [fullspec(a) for a in ws]),
        out_specs=pl.BlockSpec((blk, 4), lambda i: (i, 0)),
        out_shape=jax.ShapeDtypeStruct((N, 4), jnp.float32),
    )(feat, *ws)


def kernel(p, boundaries, planes_xy, planes_xz, planes_yz,
           c_planes_xy, c_planes_xz, c_planes_yz,
           W0, b0, W1, b1, Wout, bout, cW0, cb0, cW1, cb1, cWout, cbout):
    nb = N // 128
    px = p[:, 0].reshape(nb, 128)
    py = p[:, 1].reshape(nb, 128)
    pz = p[:, 2].reshape(nb, 128)
    iw3 = _prep(px, py, pz, boundaries)
    iw = iw3.reshape(24, N)

    tabs = [a.reshape(S * R * R, IN_DIM)
            for a in (planes_xy, planes_xz, planes_yz,
                      c_planes_xy, c_planes_xz, c_planes_yz)]
    feat = _gather_sc(iw, *tabs)

    zz = jnp.zeros((HID, HID), jnp.float32)
    W0c = jnp.block([[W0, zz], [zz, cW0]])
    b0c = jnp.concatenate([b0, cb0]).reshape(1, D2)
    W1c = jnp.block([[W1, zz], [zz, cW1]])
    b1c = jnp.concatenate([b1, cb1]).reshape(1, D2)
    Wf = jnp.block([[jnp.zeros((HID, 3), jnp.float32), Wout],
                    [cWout, jnp.zeros((HID, 1), jnp.float32)]])
    bf = jnp.concatenate([cbout, bout]).reshape(1, 4)
    return _mlp(feat, W0c, b0c, W1c, b1c, Wf, bf)


# speedup vs baseline: 1.0681x; 1.0173x over previous
import jax
import jax.numpy as jnp
from jax import lax
from jax.experimental import pallas as pl
from jax.experimental.pallas import tpu as pltpu
from jax.experimental.pallas import tpu_sc as plsc

S = 8
R = 128
IN_DIM = 32
HID = 32
N = 262144
D2 = 2 * IN_DIM

NC = 2
NS = 16
NW = NC * NS
PPW = N // NW
CH = 64
NCHUNK = PPW // CH


def _prep_body(b_ref, px_ref, py_ref, pz_ref, iw_ref):
    px = px_ref[...]
    py = py_ref[...]
    pz = pz_ref[...]
    shp = px.shape
    pre = jnp.zeros(shp, jnp.bool_)
    lox = jnp.zeros(shp, jnp.float32)
    loy = jnp.zeros(shp, jnp.float32)
    loz = jnp.zeros(shp, jnp.float32)
    hix = jnp.ones(shp, jnp.float32)
    hiy = jnp.ones(shp, jnp.float32)
    hiz = jnp.ones(shp, jnp.float32)
    sidx = jnp.zeros(shp, jnp.int32)
    for s in range(S):
        l0 = b_ref[s, 0, 0]
        l1 = b_ref[s, 0, 1]
        l2 = b_ref[s, 0, 2]
        h0 = b_ref[s, 1, 0]
        h1 = b_ref[s, 1, 1]
        h2 = b_ref[s, 1, 2]
        m = ((px > l0) & (px < h0) & (py > l1) & (py < h1)
             & (pz > l2) & (pz < h2) & (~pre))
        pre = pre | m
        lox = jnp.where(m, l0, lox)
        loy = jnp.where(m, l1, loy)
        loz = jnp.where(m, l2, loz)
        hix = jnp.where(m, h0, hix)
        hiy = jnp.where(m, h1, hiy)
        hiz = jnp.where(m, h2, hiz)
        sidx = jnp.where(m, s, sidx)
    routed = pre.astype(jnp.float32)
    dx = jnp.where(pre, hix - lox, 1.0)
    dy = jnp.where(pre, hiy - loy, 1.0)
    dz = jnp.where(pre, hiz - loz, 1.0)
    un = (px - lox) / dx
    vn = (py - loy) / dy
    tn = (pz - loz) / dz
    sbase = sidx * (R * R)
    for o, (ca, cb) in enumerate(((un, vn), (un, tn), (vn, tn))):
        xx = jnp.clip(ca, 0.0, 1.0) * (R - 1)
        yy = jnp.clip(cb, 0.0, 1.0) * (R - 1)
        x0 = jnp.clip(jnp.floor(xx), 0, R - 2).astype(jnp.int32)
        y0 = jnp.clip(jnp.floor(yy), 0, R - 2).astype(jnp.int32)
        wx = xx - x0.astype(jnp.float32)
        wy = yy - y0.astype(jnp.float32)
        base = sbase + x0 * R + y0
        iw_ref[4 * o + 0] = base
        iw_ref[4 * o + 1] = base + 1
        iw_ref[4 * o + 2] = base + R
        iw_ref[4 * o + 3] = base + R + 1
        wq = ((1 - wx) * (1 - wy), (1 - wx) * wy,
              wx * (1 - wy), wx * wy)
        for q in range(4):
            iw_ref[12 + 4 * o + q] = lax.bitcast_convert_type(
                wq[q] * routed, jnp.int32)


def _prep(px, py, pz, boundaries):
    nb = px.shape[0]
    blk = 256
    grid = nb // blk
    return pl.pallas_call(
        _prep_body,
        grid=(grid,),
        in_specs=[
            pl.BlockSpec(memory_space=pltpu.SMEM),
            pl.BlockSpec((blk, 128), lambda i: (i, 0)),
            pl.BlockSpec((blk, 128), lambda i: (i, 0)),
            pl.BlockSpec((blk, 128), lambda i: (i, 0)),
        ],
        out_specs=pl.BlockSpec((24, blk, 128), lambda i: (0, i, 0)),
        out_shape=jax.ShapeDtypeStruct((24, nb, 128), jnp.int32),
        compiler_params=pltpu.CompilerParams(
            allow_input_fusion=[False, True, True, True]),
    )(boundaries, px, py, pz)


def _sc_body(iw_hbm, t0, t1, t2, t3, t4, t5, feat_hbm, *scr):
    iw_v = scr[0:4]
    rows = (scr[4:28], scr[28:52])
    outb = scr[52:54]
    gsem = scr[54:56]
    osem = scr[56:58]
    iwsem = scr[58:62]
    tabs = (t0, t1, t2, t3, t4, t5)

    wid = lax.axis_index("s") * NC + lax.axis_index("c")
    base0 = wid * PPW
    iota16 = lax.iota(jnp.int32, 16)

    def iw_load(c, ib):
        base = pl.multiple_of(base0 + c * CH, CH)
        pltpu.sync_copy(iw_hbm.at[:, pl.ds(base, CH)], iw_v[ib])

    def iw_fire(c, ib):
        base = pl.multiple_of(base0 + c * CH, CH)
        pltpu.async_copy(iw_hbm.at[:, pl.ds(base, CH)], iw_v[ib],
                         iwsem[ib])

    def iw_drain(ib):
        pltpu.make_async_copy(iw_hbm.at[:, pl.ds(0, CH)], iw_v[ib],
                              iwsem[ib]).wait()

    def fire(b, ib):
        for j in range(24):
            pltpu.async_copy(tabs[3 * (j // 12) + (j % 12) // 4]
                             .at[iw_v[ib].at[j % 12]],
                             rows[b][j], gsem[b])

    def drain_gathers(b, ib):
        for j in range(24):
            pltpu.make_async_copy(tabs[3 * (j // 12) + (j % 12) // 4]
                                  .at[iw_v[ib].at[j % 12]],
                                  rows[b][j], gsem[b]).wait()

    kvecs = [iota16 + 16 * k for k in range(D2 // 16)]

    def compute(b, ib):
        def p_body(pi, carry):
            for u in range(2):
                p = 2 * pi + u
                pv = jnp.full((16,), 0, jnp.int32) + p
                wq = [plsc.bitcast(
                    plsc.load_gather(iw_v[ib],
                                     [jnp.full((16,), 12 + q, jnp.int32),
                                      pv]),
                    jnp.float32) for q in range(12)]
                for k in range(4):
                    half = 12 * (k // 2)
                    kv = kvecs[k % 2]
                    acc = None
                    for j in range(12):
                        t = wq[j] * plsc.load_gather(rows[b][half + j],
                                                    [pv, kv])
                        acc = t if acc is None else acc + t
                    plsc.store_scatter(outb[b], [pv, kvecs[k]], acc)
            return carry

        lax.fori_loop(0, CH // 2, p_body, 0)

    def out_fire(c, b):
        base = pl.multiple_of(base0 + c * CH, CH)
        pltpu.async_copy(outb[b], feat_hbm.at[pl.ds(base, CH)], osem[b])

    def out_drain(b):
        pltpu.make_async_copy(outb[b], feat_hbm.at[pl.ds(0, CH)],
                              osem[b]).wait()

    iw_load(0, 0)
    iw_load(1, 1)
    iw_load(2, 2)
    fire(0, 0)

    def quad_body(i, carry):
        for u in range(4):
            c = 4 * i + u
            b = u % 2
            ib = u
            nc = c + 1

            @pl.when(nc < NCHUNK)
            def _():
                @pl.when(nc >= 3)
                def _():
                    iw_drain((u + 1) % 4)
                fire(1 - b, (u + 1) % 4)

            @pl.when(c + 3 < NCHUNK)
            def _():
                iw_fire(c + 3, (u + 3) % 4)

            drain_gathers(b, ib)

            @pl.when(c >= 2)
            def _():
                out_drain(b)

            compute(b, ib)
            out_fire(c, b)
        return carry

    lax.fori_loop(0, NCHUNK // 4, quad_body, 0)
    out_drain(0)
    out_drain(1)


def _gather_sc(iw, *tabs):
    mesh = plsc.VectorSubcoreMesh(
        core_axis_name="c", subcore_axis_name="s",
        num_cores=NC, num_subcores=NS)
    scratch = (
        [pltpu.VMEM((24, CH), jnp.int32) for _ in range(4)]
        + [pltpu.VMEM((CH, IN_DIM), jnp.float32) for _ in range(48)]
        + [pltpu.VMEM((CH, D2), jnp.float32) for _ in range(2)]
        + [pltpu.SemaphoreType.DMA for _ in range(8)]
    )
    fn = pl.kernel(
        _sc_body,
        out_type=jax.ShapeDtypeStruct((N, D2), jnp.float32),
        mesh=mesh,
        scratch_types=scratch,
        compiler_params=pltpu.CompilerParams(use_tc_tiling_on_sc=False,
                                             needs_layout_passes=False),
    )
    return fn(iw, *tabs)


def _mlp_body(f_ref, W0r, b0r, W1r, b1r, Wfr, bfr, out_ref):
    f = f_ref[...]
    h = jnp.maximum(jnp.dot(f, W0r[...], preferred_element_type=jnp.float32)
                    + b0r[...], 0.0)
    h = jnp.maximum(jnp.dot(h, W1r[...], preferred_element_type=jnp.float32)
                    + b1r[...], 0.0)
    z = jnp.dot(h, Wfr[...], preferred_element_type=jnp.float32) + bfr[...]
    col = lax.broadcasted_iota(jnp.int32, z.shape, 1)
    out_ref[...] = jnp.where(col < 3, jax.nn.sigmoid(z), jnp.tanh(z))


def _mlp(feat, W0c, b0c, W1c, b1c, Wf, bf):
    blk = 8192
    grid = N // blk

    def fullspec(a):
        return pl.BlockSpec(a.shape, lambda i: (0,) * a.ndim)

    ws = [W0c, b0c, W1c, b1c, Wf, bf]
    return pl.pallas_call(
        _mlp_body,
        grid=(grid,),
        in_specs=([pl.BlockSpec((blk, D2), lambda i: (i, 0))]
                  + [fullspec(a) for a in ws]),
        out_specs=pl.BlockSpec((blk, 4), lambda i: (i, 0)),
        out_shape=jax.ShapeDtypeStruct((N, 4), jnp.float32),
    )(feat, *ws)


def kernel(p, boundaries, planes_xy, planes_xz, planes_yz,
           c_planes_xy, c_planes_xz, c_planes_yz,
           W0, b0, W1, b1, Wout, bout, cW0, cb0, cW1, cb1, cWout, cbout):
    nb = N // 128
    px = p[:, 0].reshape(nb, 128)
    py = p[:, 1].reshape(nb, 128)
    pz = p[:, 2].reshape(nb, 128)
    iw3 = _prep(px, py, pz, boundaries)
    iw = iw3.reshape(24, N)

    tabs = [a.reshape(S * R * R, IN_DIM)
            for a in (planes_xy, planes_xz, planes_yz,
                      c_planes_xy, c_planes_xz, c_planes_yz)]
    feat = _gather_sc(iw, *tabs)

    zz = jnp.zeros((HID, HID), jnp.float32)
    W0c = jnp.block([[W0, zz], [zz, cW0]])
    b0c = jnp.concatenate([b0, cb0]).reshape(1, D2)
    W1c = jnp.block([[W1, zz], [zz, cW1]])
    b1c = jnp.concatenate([b1, cb1]).reshape(1, D2)
    Wf = jnp.block([[jnp.zeros((HID, 3), jnp.float32), Wout],
                    [cWout, jnp.zeros((HID, 1), jnp.float32)]])
    bf = jnp.concatenate([cbout, bout]).reshape(1, 4)
    return _mlp(feat, W0c, b0c, W1c, b1c, Wf, bf)
